# baseline (speedup 1.0000x reference)
import dataclasses
import functools

import jax
import jax.numpy as jnp
from jax import lax
from jax.experimental import pallas as pl
from jax.experimental.pallas import tpu as pltpu
from jax.experimental.pallas import tpu_sc as plsc

N_NODES = 10000
N_EDGES = 160000
D_FEAT = 256
HIDDEN = 512
OUT = 256

_NC = 2
_NS = 16
_LANES = 16

_EPS = N_EDGES // _NS
_BATCH = 80
_NBATCH = _EPS // _BATCH
_GRP = 2
_PADN = _GRP * _BATCH
_LISTN = _EPS + _PADN + _LANES
_DUMP = _EPS + _PADN
_ROWS2D = _LISTN // _BATCH + 1
_N_PAD = 10240
_HALF = 5120
_ACC_ROWS = 5248
_ZRPS = _ACC_ROWS // _NS
_ORPS = _HALF // _NS


def _sc_compiler_params():
    cp = pltpu.CompilerParams()
    if "needs_layout_passes" in pltpu.CompilerParams.__dataclass_fields__:
        cp = dataclasses.replace(cp, needs_layout_passes=False)
    return cp


def _compact_my_half(srcF, dstF, c):
    lane = lax.iota(jnp.int32, _LANES)

    @pl.loop(0, _EPS // _LANES, init_carry=jnp.int32(0))
    def off(i, off):
        dv = dstF[pl.ds(i * _LANES, _LANES)]
        reb = dv - c * _HALF
        ok = (reb >= 0) & (reb < _HALF)
        ok_i = ok.astype(jnp.int32)
        pos = jnp.where(ok, off + plsc.cumsum(ok_i) - 1, _DUMP + lane)
        if srcF is not None:
            sv = srcF[pl.ds(i * _LANES, _LANES)]
            plsc.store_scatter(srcF, [pos], sv)
        plsc.store_scatter(dstF, [pos], reb)
        return off + jnp.sum(ok_i)

    for j in range(_PADN // _LANES):
        if srcF is not None:
            srcF[pl.ds(off + j * _LANES, _LANES)] = jnp.zeros((_LANES,),
                                                              jnp.int32)
        dstF[pl.ds(off + j * _LANES, _LANES)] = (_HALF + (j * _LANES) % 128
                                                 + lane)
    return (off + _PADN - 1) // _PADN


@functools.cache
def _make_sc_agg(n_chunks: int):
    mesh = plsc.VectorSubcoreMesh(core_axis_name="c", subcore_axis_name="s")

    def body(table, src, dst, zrow, out_agg,
             srcF, dstF, didx2d, gidx2d, rows, accum, gsem, ssem):
        c = lax.axis_index("c")
        s = lax.axis_index("s")
        myz = pl.ds(s * _ZRPS, _ZRPS)

        pltpu.sync_copy(src.at[pl.ds(s * _EPS, _EPS)], srcF.at[pl.ds(0, _EPS)])
        pltpu.sync_copy(dst.at[pl.ds(s * _EPS, _EPS)], dstF.at[pl.ds(0, _EPS)])
        pltpu.sync_copy(zrow.at[myz], accum.at[myz])

        ng = _compact_my_half(srcF, dstF, c)
        nb = ng * _GRP

        @pl.loop(0, nb)
        def _(r):
            for j in range(_BATCH // _LANES):
                sl = pl.ds(j * _LANES, _LANES)
                didx2d[r, sl] = dstF[pl.ds(r * _BATCH + j * _LANES, _LANES)]

        plsc.subcore_barrier()

        for k in range(n_chunks):
            if k:
                plsc.subcore_barrier()
                pltpu.sync_copy(zrow.at[myz], accum.at[myz])
                plsc.subcore_barrier()

            @pl.loop(0, nb)
            def _(r):
                for j in range(_BATCH // _LANES):
                    sl = pl.ds(j * _LANES, _LANES)
                    gidx2d[r, sl] = (
                        srcF[pl.ds(r * _BATCH + j * _LANES, _LANES)] * n_chunks
                        + k)

            rowsA = rows.at[pl.ds(0, _BATCH)]
            rowsB = rows.at[pl.ds(_BATCH, _BATCH)]

            @pl.loop(0, ng)
            def _(q):
                @pl.when(q > 0)
                def _():
                    pltpu.make_async_copy(rowsA, accum.at[didx2d.at[0]],
                                          ssem).wait()
                pltpu.sync_copy(table.at[gidx2d.at[2 * q]], rowsA)
                pltpu.async_copy(rowsA, accum.at[didx2d.at[2 * q]], ssem,
                                 add=True)

                @pl.when(q > 0)
                def _():
                    pltpu.make_async_copy(rowsB, accum.at[didx2d.at[0]],
                                          ssem).wait()
                pltpu.sync_copy(table.at[gidx2d.at[2 * q + 1]], rowsB)
                pltpu.async_copy(rowsB, accum.at[didx2d.at[2 * q + 1]], ssem,
                                 add=True)

            pltpu.make_async_copy(rowsA, accum.at[didx2d.at[0]], ssem).wait()
            pltpu.make_async_copy(rowsB, accum.at[didx2d.at[0]], ssem).wait()

            plsc.subcore_barrier()
            pltpu.sync_copy(
                accum.at[pl.ds(s * _ORPS, _ORPS)],
                out_agg.at[pl.ds(k * _N_PAD + c * _HALF + s * _ORPS, _ORPS)])

    return pl.kernel(
        body,
        out_type=jax.ShapeDtypeStruct((n_chunks * _N_PAD, 128), jnp.float32),
        mesh=mesh,
        scratch_types=[
            pltpu.VMEM((_LISTN,), jnp.int32),
            pltpu.VMEM((_LISTN,), jnp.int32),
            pltpu.VMEM((_ROWS2D, _BATCH), jnp.int32),
            pltpu.VMEM((_ROWS2D, _BATCH), jnp.int32),
            pltpu.VMEM((_GRP * _BATCH, 128), jnp.float32),
            pltpu.VMEM_SHARED((_ACC_ROWS, 128), jnp.float32),
            pltpu.SemaphoreType.DMA,
            pltpu.SemaphoreType.DMA,
        ],
        compiler_params=_sc_compiler_params(),
    )


@functools.cache
def _make_sc_deg():
    mesh = plsc.VectorSubcoreMesh(core_axis_name="c", subcore_axis_name="s")

    def body(dst, zrow, out_deg, dstF, didx2d, ones, dacc, ssem):
        c = lax.axis_index("c")
        s = lax.axis_index("s")
        myz = pl.ds(s * _ZRPS, _ZRPS)

        pltpu.sync_copy(dst.at[pl.ds(s * _EPS, _EPS)], dstF.at[pl.ds(0, _EPS)])
        pltpu.sync_copy(zrow.at[myz], dacc.at[myz])

        nb = _compact_my_half(None, dstF, c) * _GRP

        @pl.loop(0, nb)
        def _(r):
            for j in range(_BATCH // _LANES):
                sl = pl.ds(j * _LANES, _LANES)
                didx2d[r, sl] = dstF[pl.ds(r * _BATCH + j * _LANES, _LANES)]

        @pl.loop(0, _BATCH)
        def _(r):
            @pl.loop(0, 128 // _LANES)
            def _(j):
                ones[r, pl.ds(j * _LANES, _LANES)] = jnp.full((_LANES,), 1.0,
                                                              jnp.float32)

        plsc.subcore_barrier()

        @pl.loop(0, nb)
        def _(r):
            pltpu.sync_copy(ones, dacc.at[didx2d.at[r]], add=True)

        plsc.subcore_barrier()
        pltpu.sync_copy(dacc.at[pl.ds(s * _ORPS, _ORPS)],
                        out_deg.at[pl.ds(c * _HALF + s * _ORPS, _ORPS)])

    return pl.kernel(
        body,
        out_type=jax.ShapeDtypeStruct((_NC * _HALF, 128), jnp.float32),
        mesh=mesh,
        scratch_types=[
            pltpu.VMEM((_LISTN,), jnp.int32),
            pltpu.VMEM((_ROWS2D, _BATCH), jnp.int32),
            pltpu.VMEM((_BATCH, 128), jnp.float32),
            pltpu.VMEM_SHARED((_ACC_ROWS, 128), jnp.float32),
            pltpu.SemaphoreType.DMA,
        ],
        compiler_params=_sc_compiler_params(),
    )


_ROWBLK = 1000


def _tc_layer(x, agg, degw, W, b, d_in, d_out, n_chunks):
    def body(x_ref, a_ref, d_ref, w_ref, b_ref, o_ref):
        recip = 1.0 / jnp.maximum(d_ref[:, 0:1], 1.0)
        acc = jnp.dot(x_ref[...], w_ref[0:d_in, :],
                      preferred_element_type=jnp.float32)
        for k in range(n_chunks):
            nk = a_ref[k] * recip
            acc = acc + jnp.dot(nk, w_ref[d_in + 128 * k:d_in + 128 * (k + 1), :],
                                preferred_element_type=jnp.float32)
        o_ref[...] = jnp.maximum(acc + b_ref[...], 0.0)

    return pl.pallas_call(
        body,
        grid=(N_NODES // _ROWBLK,),
        in_specs=[
            pl.BlockSpec((_ROWBLK, d_in), lambda i: (i, 0)),
            pl.BlockSpec((n_chunks, _ROWBLK, 128), lambda i: (0, i, 0)),
            pl.BlockSpec((_ROWBLK, 128), lambda i: (i, 0)),
            pl.BlockSpec((2 * d_in, d_out), lambda i: (0, 0)),
            pl.BlockSpec((1, d_out), lambda i: (0, 0)),
        ],
        out_specs=pl.BlockSpec((_ROWBLK, d_out), lambda i: (i, 0)),
        out_shape=jax.ShapeDtypeStruct((N_NODES, d_out), jnp.float32),
        compiler_params=pltpu.CompilerParams(
            dimension_semantics=("parallel",)),
    )(x, agg, degw, W, b)


def kernel(x, edge_index, W1, b1, W2, b2):
    src = edge_index[0]
    dst = edge_index[1]
    zrow = jnp.zeros((_ACC_ROWS, 128), jnp.float32)

    degw = _make_sc_deg()(dst, zrow)

    nc1 = D_FEAT // 128
    nc2 = HIDDEN // 128
    agg1 = _make_sc_agg(nc1)(
        x.reshape(nc1 * N_NODES, 128), src, dst, zrow)
    h = _tc_layer(x, agg1.reshape(nc1, _N_PAD, 128), degw,
                  W1, b1.reshape(1, HIDDEN), D_FEAT, HIDDEN, nc1)

    agg2 = _make_sc_agg(nc2)(
        h.reshape(nc2 * N_NODES, 128), src, dst, zrow)
    out = _tc_layer(h, agg2.reshape(nc2, _N_PAD, 128), degw,
                    W2, b2.reshape(1, OUT), HIDDEN, OUT, nc2)
    return out


# speedup vs baseline: 1.1702x; 1.1702x over previous
import dataclasses
import functools

import jax
import jax.numpy as jnp
from jax import lax
from jax.experimental import pallas as pl
from jax.experimental.pallas import tpu as pltpu
from jax.experimental.pallas import tpu_sc as plsc

N_NODES = 10000
N_EDGES = 160000
D_FEAT = 256
HIDDEN = 512
OUT = 256

_NC = 2
_NS = 16
_LANES = 16

_EPS = N_EDGES // _NS
_BATCH = 80
_NBATCH = _EPS // _BATCH
_GRP = 1
_PADN = _GRP * _BATCH
_LISTN = _EPS + _PADN + _LANES
_DUMP = _EPS + _PADN
_ROWS2D = _LISTN // _BATCH + 1
_N_PAD = 10240
_HALF = 5120
_ACC_ROWS = 5248
_ZRPS = _ACC_ROWS // _NS
_ORPS = _HALF // _NS


def _sc_compiler_params():
    cp = pltpu.CompilerParams()
    if "needs_layout_passes" in pltpu.CompilerParams.__dataclass_fields__:
        cp = dataclasses.replace(cp, needs_layout_passes=False)
    return cp


def _compact_my_half(srcF, dstF, c):
    lane = lax.iota(jnp.int32, _LANES)

    @pl.loop(0, _EPS // _LANES, init_carry=jnp.int32(0))
    def off(i, off):
        dv = dstF[pl.ds(i * _LANES, _LANES)]
        reb = dv - c * _HALF
        ok = (reb >= 0) & (reb < _HALF)
        ok_i = ok.astype(jnp.int32)
        pos = jnp.where(ok, off + plsc.cumsum(ok_i) - 1, _DUMP + lane)
        if srcF is not None:
            sv = srcF[pl.ds(i * _LANES, _LANES)]
            plsc.store_scatter(srcF, [pos], sv)
        plsc.store_scatter(dstF, [pos], reb)
        return off + jnp.sum(ok_i)

    for j in range(_PADN // _LANES):
        if srcF is not None:
            srcF[pl.ds(off + j * _LANES, _LANES)] = jnp.zeros((_LANES,),
                                                              jnp.int32)
        dstF[pl.ds(off + j * _LANES, _LANES)] = (_HALF + (j * _LANES) % 128
                                                 + lane)
    return (off + _PADN - 1) // _PADN


@functools.cache
def _make_sc_agg(n_chunks: int):
    mesh = plsc.VectorSubcoreMesh(core_axis_name="c", subcore_axis_name="s")

    def body(table, src, dst, zrow, out_agg,
             srcF, dstF, didx2d, gidx2d, rows, accum, gsem, ssem):
        c = lax.axis_index("c")
        s = lax.axis_index("s")
        myz = pl.ds(s * _ZRPS, _ZRPS)

        pltpu.sync_copy(src.at[pl.ds(s * _EPS, _EPS)], srcF.at[pl.ds(0, _EPS)])
        pltpu.sync_copy(dst.at[pl.ds(s * _EPS, _EPS)], dstF.at[pl.ds(0, _EPS)])
        pltpu.sync_copy(zrow.at[myz], accum.at[myz])

        ng = _compact_my_half(srcF, dstF, c)
        nb = ng * _GRP

        @pl.loop(0, nb)
        def _(r):
            for j in range(_BATCH // _LANES):
                sl = pl.ds(j * _LANES, _LANES)
                didx2d[r, sl] = dstF[pl.ds(r * _BATCH + j * _LANES, _LANES)]

        plsc.subcore_barrier()

        for k in range(n_chunks):
            if k:
                plsc.subcore_barrier()
                pltpu.sync_copy(zrow.at[myz], accum.at[myz])
                plsc.subcore_barrier()

            @pl.loop(0, nb)
            def _(r):
                for j in range(_BATCH // _LANES):
                    sl = pl.ds(j * _LANES, _LANES)
                    gidx2d[r, sl] = (
                        srcF[pl.ds(r * _BATCH + j * _LANES, _LANES)] * n_chunks
                        + k)

            @pl.loop(0, nb)
            def _(r):
                pltpu.sync_copy(table.at[gidx2d.at[r]], rows)
                pltpu.sync_copy(rows, accum.at[didx2d.at[r]], add=True)

            plsc.subcore_barrier()
            pltpu.sync_copy(
                accum.at[pl.ds(s * _ORPS, _ORPS)],
                out_agg.at[pl.ds(k * _N_PAD + c * _HALF + s * _ORPS, _ORPS)])

    return pl.kernel(
        body,
        out_type=jax.ShapeDtypeStruct((n_chunks * _N_PAD, 128), jnp.float32),
        mesh=mesh,
        scratch_types=[
            pltpu.VMEM((_LISTN,), jnp.int32),
            pltpu.VMEM((_LISTN,), jnp.int32),
            pltpu.VMEM((_ROWS2D, _BATCH), jnp.int32),
            pltpu.VMEM((_ROWS2D, _BATCH), jnp.int32),
            pltpu.VMEM((_GRP * _BATCH, 128), jnp.float32),
            pltpu.VMEM_SHARED((_ACC_ROWS, 128), jnp.float32),
            pltpu.SemaphoreType.DMA,
            pltpu.SemaphoreType.DMA,
        ],
        compiler_params=_sc_compiler_params(),
    )


@functools.cache
def _make_sc_deg():
    mesh = plsc.VectorSubcoreMesh(core_axis_name="c", subcore_axis_name="s")

    def body(dst, zrow, out_deg, dstF, didx2d, ones, dacc, ssem):
        c = lax.axis_index("c")
        s = lax.axis_index("s")
        myz = pl.ds(s * _ZRPS, _ZRPS)

        pltpu.sync_copy(dst.at[pl.ds(s * _EPS, _EPS)], dstF.at[pl.ds(0, _EPS)])
        pltpu.sync_copy(zrow.at[myz], dacc.at[myz])

        nb = _compact_my_half(None, dstF, c) * _GRP

        @pl.loop(0, nb)
        def _(r):
            for j in range(_BATCH // _LANES):
                sl = pl.ds(j * _LANES, _LANES)
                didx2d[r, sl] = dstF[pl.ds(r * _BATCH + j * _LANES, _LANES)]

        @pl.loop(0, _BATCH)
        def _(r):
            @pl.loop(0, 128 // _LANES)
            def _(j):
                ones[r, pl.ds(j * _LANES, _LANES)] = jnp.full((_LANES,), 1.0,
                                                              jnp.float32)

        plsc.subcore_barrier()

        @pl.loop(0, nb)
        def _(r):
            pltpu.sync_copy(ones, dacc.at[didx2d.at[r]], add=True)

        plsc.subcore_barrier()
        pltpu.sync_copy(dacc.at[pl.ds(s * _ORPS, _ORPS)],
                        out_deg.at[pl.ds(c * _HALF + s * _ORPS, _ORPS)])

    return pl.kernel(
        body,
        out_type=jax.ShapeDtypeStruct((_NC * _HALF, 128), jnp.float32),
        mesh=mesh,
        scratch_types=[
            pltpu.VMEM((_LISTN,), jnp.int32),
            pltpu.VMEM((_ROWS2D, _BATCH), jnp.int32),
            pltpu.VMEM((_BATCH, 128), jnp.float32),
            pltpu.VMEM_SHARED((_ACC_ROWS, 128), jnp.float32),
            pltpu.SemaphoreType.DMA,
        ],
        compiler_params=_sc_compiler_params(),
    )


_ROWBLK = 1000


def _tc_self(x, W_top, b, d_in, d_out):
    def body(x_ref, w_ref, b_ref, o_ref):
        o_ref[...] = jnp.dot(x_ref[...], w_ref[...],
                             preferred_element_type=jnp.float32) + b_ref[...]

    return pl.pallas_call(
        body,
        grid=(N_NODES // _ROWBLK,),
        in_specs=[
            pl.BlockSpec((_ROWBLK, d_in), lambda i: (i, 0)),
            pl.BlockSpec((d_in, d_out), lambda i: (0, 0)),
            pl.BlockSpec((1, d_out), lambda i: (0, 0)),
        ],
        out_specs=pl.BlockSpec((_ROWBLK, d_out), lambda i: (i, 0)),
        out_shape=jax.ShapeDtypeStruct((N_NODES, d_out), jnp.float32),
        compiler_params=pltpu.CompilerParams(
            dimension_semantics=("parallel",)),
    )(x, W_top, b)


def _tc_combine(p, agg, degw, W, d_in, d_out, n_chunks):
    def body(p_ref, a_ref, d_ref, w_ref, o_ref):
        recip = 1.0 / jnp.maximum(d_ref[:, 0:1], 1.0)
        acc = p_ref[...]
        for k in range(n_chunks):
            nk = a_ref[k] * recip
            acc = acc + jnp.dot(nk, w_ref[d_in + 128 * k:d_in + 128 * (k + 1), :],
                                preferred_element_type=jnp.float32)
        o_ref[...] = jnp.maximum(acc, 0.0)

    return pl.pallas_call(
        body,
        grid=(N_NODES // _ROWBLK,),
        in_specs=[
            pl.BlockSpec((_ROWBLK, d_out), lambda i: (i, 0)),
            pl.BlockSpec((n_chunks, _ROWBLK, 128), lambda i: (0, i, 0)),
            pl.BlockSpec((_ROWBLK, 128), lambda i: (i, 0)),
            pl.BlockSpec((2 * d_in, d_out), lambda i: (0, 0)),
        ],
        out_specs=pl.BlockSpec((_ROWBLK, d_out), lambda i: (i, 0)),
        out_shape=jax.ShapeDtypeStruct((N_NODES, d_out), jnp.float32),
        compiler_params=pltpu.CompilerParams(
            dimension_semantics=("parallel",)),
    )(p, agg, degw, W)


def kernel(x, edge_index, W1, b1, W2, b2):
    src = edge_index[0]
    dst = edge_index[1]
    zrow = jnp.zeros((_ACC_ROWS, 128), jnp.float32)

    degw = _make_sc_deg()(dst, zrow)

    nc1 = D_FEAT // 128
    nc2 = HIDDEN // 128
    agg1 = _make_sc_agg(nc1)(
        x.reshape(nc1 * N_NODES, 128), src, dst, zrow)
    p1 = _tc_self(x, W1[:D_FEAT], b1.reshape(1, HIDDEN), D_FEAT, HIDDEN)
    h = _tc_combine(p1, agg1.reshape(nc1, _N_PAD, 128), degw,
                    W1, D_FEAT, HIDDEN, nc1)

    agg2 = _make_sc_agg(nc2)(
        h.reshape(nc2 * N_NODES, 128), src, dst, zrow)
    p2 = _tc_self(h, W2[:HIDDEN], b2.reshape(1, OUT), HIDDEN, OUT)
    out = _tc_combine(p2, agg2.reshape(nc2, _N_PAD, 128), degw,
                      W2, HIDDEN, OUT, nc2)
    return out


# speedup vs baseline: 1.1766x; 1.0055x over previous
import dataclasses
import functools

import jax
import jax.numpy as jnp
from jax import lax
from jax.experimental import pallas as pl
from jax.experimental.pallas import tpu as pltpu
from jax.experimental.pallas import tpu_sc as plsc

N_NODES = 10000
N_EDGES = 160000
D_FEAT = 256
HIDDEN = 512
OUT = 256

_NC = 2
_NS = 16
_LANES = 16

_EPS = N_EDGES // _NS
_BATCH = 80
_NBATCH = _EPS // _BATCH
_GRP = 1
_PADN = _GRP * _BATCH
_LISTN = _EPS + _PADN + _LANES
_DUMP = _EPS + _PADN
_ROWS2D = _LISTN // _BATCH + 1
_N_PAD = 10240
_HALF = 5120
_ACC_ROWS = 5248
_ZRPS = _ACC_ROWS // _NS
_ORPS = _HALF // _NS


def _sc_compiler_params():
    cp = pltpu.CompilerParams()
    if "needs_layout_passes" in pltpu.CompilerParams.__dataclass_fields__:
        cp = dataclasses.replace(cp, needs_layout_passes=False)
    return cp


def _compact_my_half(srcF, dstF, c):
    lane = lax.iota(jnp.int32, _LANES)

    @pl.loop(0, _EPS // _LANES, init_carry=jnp.int32(0))
    def off(i, off):
        dv = dstF[pl.ds(i * _LANES, _LANES)]
        reb = dv - c * _HALF
        ok = (reb >= 0) & (reb < _HALF)
        ok_i = ok.astype(jnp.int32)
        pos = jnp.where(ok, off + plsc.cumsum(ok_i) - 1, _DUMP + lane)
        if srcF is not None:
            sv = srcF[pl.ds(i * _LANES, _LANES)]
            plsc.store_scatter(srcF, [pos], sv)
        plsc.store_scatter(dstF, [pos], reb)
        return off + jnp.sum(ok_i)

    for j in range(_PADN // _LANES):
        if srcF is not None:
            srcF[pl.ds(off + j * _LANES, _LANES)] = jnp.zeros((_LANES,),
                                                              jnp.int32)
        dstF[pl.ds(off + j * _LANES, _LANES)] = (_HALF + (j * _LANES) % 128
                                                 + lane)
    return (off + _PADN - 1) // _PADN


@functools.cache
def _make_sc_agg(n_chunks: int):
    mesh = plsc.VectorSubcoreMesh(core_axis_name="c", subcore_axis_name="s")

    def body(table, src, dst, zrow, out_agg,
             srcF, dstF, didx2d, gidx2d, rows, accum, gsem, ssem):
        c = lax.axis_index("c")
        s = lax.axis_index("s")
        myz = pl.ds(s * _ZRPS, _ZRPS)

        pltpu.sync_copy(src.at[pl.ds(s * _EPS, _EPS)], srcF.at[pl.ds(0, _EPS)])
        pltpu.sync_copy(dst.at[pl.ds(s * _EPS, _EPS)], dstF.at[pl.ds(0, _EPS)])
        pltpu.sync_copy(zrow.at[myz], accum.at[myz])

        ng = _compact_my_half(srcF, dstF, c)
        nb = ng * _GRP

        @pl.loop(0, nb)
        def _(r):
            for j in range(_BATCH // _LANES):
                sl = pl.ds(j * _LANES, _LANES)
                didx2d[r, sl] = dstF[pl.ds(r * _BATCH + j * _LANES, _LANES)]

        plsc.subcore_barrier()

        for k in range(n_chunks):
            if k:
                plsc.subcore_barrier()
                pltpu.sync_copy(zrow.at[myz], accum.at[myz])
                plsc.subcore_barrier()

            @pl.loop(0, nb)
            def _(r):
                for j in range(_BATCH // _LANES):
                    sl = pl.ds(j * _LANES, _LANES)
                    gidx2d[r, sl] = (
                        srcF[pl.ds(r * _BATCH + j * _LANES, _LANES)] * n_chunks
                        + k)

            @pl.loop(0, nb)
            def _(r):
                pltpu.sync_copy(table.at[gidx2d.at[r]], rows)
                pltpu.sync_copy(rows, accum.at[didx2d.at[r]], add=True)

            plsc.subcore_barrier()
            pltpu.sync_copy(
                accum.at[pl.ds(s * _ORPS, _ORPS)],
                out_agg.at[pl.ds(k * _N_PAD + c * _HALF + s * _ORPS, _ORPS)])

    return pl.kernel(
        body,
        out_type=jax.ShapeDtypeStruct((n_chunks * _N_PAD, 128), jnp.float32),
        mesh=mesh,
        scratch_types=[
            pltpu.VMEM((_LISTN,), jnp.int32),
            pltpu.VMEM((_LISTN,), jnp.int32),
            pltpu.VMEM((_ROWS2D, _BATCH), jnp.int32),
            pltpu.VMEM((_ROWS2D, _BATCH), jnp.int32),
            pltpu.VMEM((_GRP * _BATCH, 128), jnp.float32),
            pltpu.VMEM_SHARED((_ACC_ROWS, 128), jnp.float32),
            pltpu.SemaphoreType.DMA,
            pltpu.SemaphoreType.DMA,
        ],
        compiler_params=_sc_compiler_params(),
    )


@functools.cache
def _make_sc_deg():
    mesh = plsc.VectorSubcoreMesh(core_axis_name="c", subcore_axis_name="s")

    def body(dst, zrow, out_deg, dstF, didx2d, ones, dacc, ssem):
        c = lax.axis_index("c")
        s = lax.axis_index("s")
        myz = pl.ds(s * _ZRPS, _ZRPS)

        pltpu.sync_copy(dst.at[pl.ds(s * _EPS, _EPS)], dstF.at[pl.ds(0, _EPS)])
        pltpu.sync_copy(zrow.at[myz], dacc.at[myz])

        nb = _compact_my_half(None, dstF, c) * _GRP

        @pl.loop(0, nb)
        def _(r):
            for j in range(_BATCH // _LANES):
                sl = pl.ds(j * _LANES, _LANES)
                didx2d[r, sl] = dstF[pl.ds(r * _BATCH + j * _LANES, _LANES)]

        @pl.loop(0, _BATCH)
        def _(r):
            @pl.loop(0, 128 // _LANES)
            def _(j):
                ones[r, pl.ds(j * _LANES, _LANES)] = jnp.full((_LANES,), 1.0,
                                                              jnp.float32)

        plsc.subcore_barrier()

        @pl.loop(0, nb)
        def _(r):
            pltpu.sync_copy(ones, dacc.at[didx2d.at[r]], add=True)

        plsc.subcore_barrier()
        pltpu.sync_copy(dacc.at[pl.ds(s * _ORPS, _ORPS)],
                        out_deg.at[pl.ds(c * _HALF + s * _ORPS, _ORPS)])

    return pl.kernel(
        body,
        out_type=jax.ShapeDtypeStruct((_NC * _HALF, 128), jnp.float32),
        mesh=mesh,
        scratch_types=[
            pltpu.VMEM((_LISTN,), jnp.int32),
            pltpu.VMEM((_ROWS2D, _BATCH), jnp.int32),
            pltpu.VMEM((_BATCH, 128), jnp.float32),
            pltpu.VMEM_SHARED((_ACC_ROWS, 128), jnp.float32),
            pltpu.SemaphoreType.DMA,
        ],
        compiler_params=_sc_compiler_params(),
    )


_ROWBLK = 1000


def _tc_layer(x, agg, degw, W, b, d_in, d_out, n_chunks):
    def body(x_ref, a_ref, d_ref, w_ref, b_ref, o_ref):
        recip = 1.0 / jnp.maximum(d_ref[:, 0:1], 1.0)
        acc = jnp.dot(x_ref[...], w_ref[0:d_in, :],
                      preferred_element_type=jnp.float32)
        for k in range(n_chunks):
            nk = a_ref[k] * recip
            acc = acc + jnp.dot(nk, w_ref[d_in + 128 * k:d_in + 128 * (k + 1), :],
                                preferred_element_type=jnp.float32)
        o_ref[...] = jnp.maximum(acc + b_ref[...], 0.0)

    return pl.pallas_call(
        body,
        grid=(N_NODES // _ROWBLK,),
        in_specs=[
            pl.BlockSpec((_ROWBLK, d_in), lambda i: (i, 0)),
            pl.BlockSpec((n_chunks, _ROWBLK, 128), lambda i: (0, i, 0)),
            pl.BlockSpec((_ROWBLK, 128), lambda i: (i, 0)),
            pl.BlockSpec((2 * d_in, d_out), lambda i: (0, 0)),
            pl.BlockSpec((1, d_out), lambda i: (0, 0)),
        ],
        out_specs=pl.BlockSpec((_ROWBLK, d_out), lambda i: (i, 0)),
        out_shape=jax.ShapeDtypeStruct((N_NODES, d_out), jnp.float32),
        compiler_params=pltpu.CompilerParams(
            dimension_semantics=("parallel",)),
    )(x, agg, degw, W, b)


def kernel(x, edge_index, W1, b1, W2, b2):
    src = edge_index[0]
    dst = edge_index[1]
    zrow = jnp.zeros((_ACC_ROWS, 128), jnp.float32)

    degw = _make_sc_deg()(dst, zrow)

    nc1 = D_FEAT // 128
    nc2 = HIDDEN // 128
    agg1 = _make_sc_agg(nc1)(
        x.reshape(nc1 * N_NODES, 128), src, dst, zrow)
    h = _tc_layer(x, agg1.reshape(nc1, _N_PAD, 128), degw,
                  W1, b1.reshape(1, HIDDEN), D_FEAT, HIDDEN, nc1)

    agg2 = _make_sc_agg(nc2)(
        h.reshape(nc2 * N_NODES, 128), src, dst, zrow)
    out = _tc_layer(h, agg2.reshape(nc2, _N_PAD, 128), degw,
                    W2, b2.reshape(1, OUT), HIDDEN, OUT, nc2)
    return out
